# speedup vs baseline: 13.2083x; 13.2083x over previous
import jax
import jax.numpy as jnp
from jax.experimental import pallas as pl

_VP = 128
_B = 2000


def _body(x_ref, t_ref, o_ref):
    xb = x_ref[...]
    f = x_ref.shape[1]
    parts = []
    for i in range(f):
        ci = jax.lax.broadcasted_iota(jnp.int32, (xb.shape[0], _VP), 1)
        parts.append((xb[:, i][:, None] == ci).astype(jnp.bfloat16))
    oh = jnp.concatenate(parts, axis=1)
    o_ref[...] = jnp.dot(oh, t_ref[...], preferred_element_type=jnp.float32)


def kernel(x, tables):
    if x.ndim == 1:
        x = x[:, None]
    n, f = x.shape
    _, v, h = tables.shape
    x = x.astype(jnp.int32)
    tp = jnp.pad(tables, ((0, 0), (0, _VP - v), (0, 0)))
    tp = tp.astype(jnp.bfloat16).reshape(f * _VP, h)
    return pl.pallas_call(
        _body,
        grid=(n // _B,),
        in_specs=[
            pl.BlockSpec((_B, f), lambda i: (i, 0)),
            pl.BlockSpec((f * _VP, h), lambda i: (0, 0)),
        ],
        out_specs=pl.BlockSpec((_B, h), lambda i: (i, 0)),
        out_shape=jax.ShapeDtypeStruct((n, h), jnp.float32),
    )(x, tp)


# speedup vs baseline: 14.1929x; 1.0745x over previous
import jax
import jax.numpy as jnp
from jax.experimental import pallas as pl

_VP = 128
_B = 2000


def _body(x_ref, s_ref, t_ref, o_ref):
    b, f = x_ref.shape
    k = s_ref.shape[1]
    xrep = jnp.dot(x_ref[...], s_ref[...],
                   preferred_element_type=jnp.float32).astype(jnp.bfloat16)
    ci = jax.lax.broadcasted_iota(jnp.int32, (8, k), 1)
    cmod = (ci & (_VP - 1)).astype(jnp.bfloat16)
    x3 = xrep.reshape(b // 8, 8, k)
    oh = jnp.where(x3 == cmod[None], jnp.bfloat16(1), jnp.bfloat16(0))
    o_ref[...] = jnp.dot(oh.reshape(b, k), t_ref[...],
                         preferred_element_type=jnp.float32)


def kernel(x, tables):
    if x.ndim == 1:
        x = x[:, None]
    n, f = x.shape
    _, v, h = tables.shape
    xb = x.astype(jnp.bfloat16)
    sel = jnp.repeat(jnp.eye(f, dtype=jnp.bfloat16), _VP, axis=1)
    tp = jnp.pad(tables, ((0, 0), (0, _VP - v), (0, 0)))
    tp = tp.astype(jnp.bfloat16).reshape(f * _VP, h)
    return pl.pallas_call(
        _body,
        grid=(n // _B,),
        in_specs=[
            pl.BlockSpec((_B, f), lambda i: (i, 0)),
            pl.BlockSpec((f, f * _VP), lambda i: (0, 0)),
            pl.BlockSpec((f * _VP, h), lambda i: (0, 0)),
        ],
        out_specs=pl.BlockSpec((_B, h), lambda i: (i, 0)),
        out_shape=jax.ShapeDtypeStruct((n, h), jnp.float32),
    )(xb, sel, tp)


# speedup vs baseline: 26.0605x; 1.8362x over previous
import jax
import jax.numpy as jnp
from jax.experimental import pallas as pl

_VP = 128
_B = 2000


def _body(xt_ref, t_ref, o_ref):
    _, f, b = xt_ref.shape
    k = t_ref.shape[0]
    xt = xt_ref[0]
    riota = jax.lax.broadcasted_iota(jnp.int32, (_VP, b), 0).astype(jnp.bfloat16)
    parts = []
    for i in range(f):
        parts.append(
            jnp.where(xt[i][None, :] == riota, jnp.bfloat16(1), jnp.bfloat16(0))
        )
    oht = jnp.concatenate(parts, axis=0)
    o_ref[...] = jax.lax.dot_general(
        oht, t_ref[...],
        dimension_numbers=(((0,), (0,)), ((), ())),
        preferred_element_type=jnp.float32,
    )


def kernel(x, tables):
    if x.ndim == 1:
        x = x[:, None]
    n, f = x.shape
    _, v, h = tables.shape
    nb = n // _B
    xt = x.T.astype(jnp.bfloat16).reshape(f, nb, _B).transpose(1, 0, 2)
    tp = jnp.pad(tables, ((0, 0), (0, _VP - v), (0, 0)))
    tp = tp.astype(jnp.bfloat16).reshape(f * _VP, h)
    return pl.pallas_call(
        _body,
        grid=(n // _B,),
        in_specs=[
            pl.BlockSpec((1, f, _B), lambda i: (i, 0, 0)),
            pl.BlockSpec((f * _VP, h), lambda i: (0, 0)),
        ],
        out_specs=pl.BlockSpec((_B, h), lambda i: (i, 0)),
        out_shape=jax.ShapeDtypeStruct((n, h), jnp.float32),
    )(xt, tp)
